# baseline (speedup 1.0000x reference)
import functools

import jax
import jax.numpy as jnp
from jax import lax
from jax.experimental import pallas as pl
from jax.experimental.pallas import tpu as pltpu
from jax.experimental.pallas import tpu_sc as plsc

N = 100000
D = 128
B = 4096
H = 256

NC = 2
NS = 16
NW = NC * NS

C = 128
FULL_CHUNKS = N // C
TAIL0 = FULL_CHUNKS * C
TAIL_ROWS = N - TAIL0
BASE_CH = FULL_CHUNKS // NW
REM_CH = FULL_CHUNKS - BASE_CH * NW
MAXM = BASE_CH + 1
NBUF = 3
_BN_INV = 1.0 / (1.0 + 1e-5) ** 0.5


def _sc_body(x_hbm, ids_hbm, sm_hbm, waw_hbm, baw_hbm,
             partial_hbm, wout_hbm,
             xv0, xv1, xv2, idv0, idv1, idv2, smv0, smv1, smv2,
             wv, idt, wawv, bawv, zv, acc,
             sin0, sin1, sin2, ssc0, ssc1, ssc2, semw, semz):
    c = lax.axis_index("c")
    s = lax.axis_index("s")
    wid = s * NC + c
    xvs = (xv0, xv1, xv2)
    idvs = (idv0, idv1, idv2)
    smvs = (smv0, smv1, smv2)
    sins = (sin0, sin1, sin2)
    sscs = (ssc0, ssc1, ssc2)

    pltpu.sync_copy(waw_hbm, wawv)
    pltpu.sync_copy(baw_hbm, bawv)

    zf = jnp.zeros((16,), jnp.float32)
    for i in range(16):
        for j in range(8):
            zv[i, pl.ds(16 * j, 16)] = zf
    for i in range(16):
        pltpu.async_copy(zv, acc.at[pl.ds(s * 256 + 16 * i, 16)], semz)

    def _copies(m, b):
        rb = (wid + NW * m) * C
        return [(x_hbm.at[pl.ds(rb, C)], xvs[b]),
                (sm_hbm.at[pl.ds(rb, C)], smvs[b]),
                (ids_hbm.at[pl.ds(rb, C)], idvs[b].at[0])]

    def _fire_in(m, b):
        for src, dst in _copies(m, b):
            pltpu.async_copy(src, dst, sins[b])

    def _wait_in(m, b):
        for src, dst in _copies(m, b):
            pltpu.make_async_copy(src, dst, sins[b]).wait()

    def _wait_sc(b):
        pltpu.make_async_copy(xvs[b], acc.at[idvs[b].at[0]], sscs[b]).wait()

    lane = lax.iota(jnp.int32, 16)

    nch = jnp.where(wid < REM_CH, 1, 0) + BASE_CH

    def _zwait(i, carry):
        pltpu.make_async_copy(zv, acc.at[pl.ds(s * 256 + 16 * i, 16)],
                              semz).wait()
        return carry
    lax.fori_loop(0, 16, _zwait, 0)
    plsc.subcore_barrier()

    _fire_in(0, 0)
    _fire_in(1, 1)

    def _process(m, b):
        _wait_in(m, b)
        xv, smv = xvs[b], smvs[b]

        @plsc.parallel_loop(0, C // 16, unroll=2)
        def _group_body(t):
            r0 = t * 16
            ww = [wawv[pl.ds(16 * j, 16)] for j in range(8)]
            bvec = bawv[...]
            smvec = smv[pl.ds(r0, 16)]
            wvec = zf
            for i in range(16):
                r = r0 + i
                vj = [xv[r, pl.ds(16 * j, 16)] for j in range(8)]
                pr = [vj[j] * ww[j] for j in range(8)]
                a = ((pr[0] + pr[1]) + (pr[2] + pr[3])) \
                    + ((pr[4] + pr[5]) + (pr[6] + pr[7]))
                sdot = jnp.sum(a)
                sv = sdot + bvec
                sg = 1.0 / (1.0 + jnp.exp(-sv))
                wn = sg * smvec[i]
                wvec = jnp.where(lane == i, wn, wvec)
                for j in range(8):
                    xv[r, pl.ds(16 * j, 16)] = vj[j] * wn
            wv[pl.ds(m * C + r0, 16)] = wvec
        pltpu.async_copy(wv.at[pl.ds(m * C, C)],
                         wout_hbm.at[pl.ds((wid + NW * m) * C, C)], semw)
        nb = (m + 2) - ((m + 2) // NBUF) * NBUF

        @pl.when((m + 2 < nch) & (m >= 1))
        def _wsc():
            for bb in range(NBUF):
                @pl.when(nb == bb)
                def _w():
                    _wait_sc(bb)

        @pl.when(m + 2 < nch)
        def _pf():
            for bb in range(NBUF):
                @pl.when(nb == bb)
                def _f():
                    _fire_in(m + 2, bb)
        pltpu.async_copy(xvs[b], acc.at[idvs[b].at[0]], sscs[b], add=True)

    def _outer(k3, carry):
        for b in range(NBUF):
            m = NBUF * k3 + b

            @pl.when(m < nch)
            def _sub():
                _process(m, b)
        return carry
    lax.fori_loop(0, (MAXM + NBUF - 1) // NBUF, _outer, 0)

    @pl.when(wid < REM_CH)
    def _dr1():
        _wait_sc(1)
        _wait_sc(2)
        _wait_sc(0)

    @pl.when(wid >= REM_CH)
    def _dr2():
        _wait_sc(0)
        _wait_sc(1)
        _wait_sc(2)

    def _wdrain(m, carry):
        pltpu.make_async_copy(wv.at[pl.ds(m * C, C)],
                              wout_hbm.at[pl.ds((wid + NW * m) * C, C)],
                              semw).wait()
        return carry
    lax.fori_loop(0, nch, _wdrain, 0)

    @pl.when(wid == NW - 1)
    def _tail():
        def _zrow(r, carry):
            for j in range(8):
                xv0[r, pl.ds(16 * j, 16)] = zf
            return carry
        lax.fori_loop(TAIL_ROWS, C, _zrow, 0)
        zi = jnp.zeros((16,), jnp.int32)
        for j in range(8):
            idv0[0, pl.ds(16 * j, 16)] = zi
        pltpu.sync_copy(x_hbm.at[pl.ds(TAIL0, TAIL_ROWS)],
                        xv0.at[pl.ds(0, TAIL_ROWS)])
        pltpu.sync_copy(ids_hbm.at[pl.ds(TAIL0, TAIL_ROWS)], idt)
        for j in range(TAIL_ROWS // 16):
            idv0[0, pl.ds(16 * j, 16)] = idt[pl.ds(16 * j, 16)]
        pltpu.sync_copy(sm_hbm.at[pl.ds(TAIL0, TAIL_ROWS)],
                        smv0.at[pl.ds(0, TAIL_ROWS)])

        def _tgroup(t, carry):
            r0 = t * 16
            ww = [wawv[pl.ds(16 * j, 16)] for j in range(8)]
            bvec = bawv[...]
            smvec = smv0[pl.ds(r0, 16)]
            wvec = zf
            for i in range(16):
                r = r0 + i
                vj = [xv0[r, pl.ds(16 * j, 16)] for j in range(8)]
                pr = [vj[j] * ww[j] for j in range(8)]
                a = ((pr[0] + pr[1]) + (pr[2] + pr[3])) \
                    + ((pr[4] + pr[5]) + (pr[6] + pr[7]))
                sdot = jnp.sum(a)
                sv = sdot + bvec
                sg = 1.0 / (1.0 + jnp.exp(-sv))
                wn = sg * smvec[i]
                wvec = jnp.where(lane == i, wn, wvec)
                for j in range(8):
                    xv0[r, pl.ds(16 * j, 16)] = vj[j] * wn
            wv[pl.ds(r0, 16)] = wvec
            return carry
        lax.fori_loop(0, TAIL_ROWS // 16, _tgroup, 0)
        pltpu.sync_copy(wv.at[pl.ds(0, TAIL_ROWS)],
                        wout_hbm.at[pl.ds(TAIL0, TAIL_ROWS)])
        pltpu.sync_copy(xv0, acc.at[idv0.at[0]], add=True)

    plsc.subcore_barrier()
    pltpu.sync_copy(acc.at[pl.ds(s * 256, 128)], xv0)
    pltpu.sync_copy(acc.at[pl.ds(s * 256 + 128, 128)], xv1)
    pltpu.sync_copy(xv0, partial_hbm.at[c, pl.ds(s * 256, 128)])
    pltpu.sync_copy(xv1, partial_hbm.at[c, pl.ds(s * 256 + 128, 128)])


_sc_call = pl.kernel(
    _sc_body,
    out_type=(
        jax.ShapeDtypeStruct((NC, B, D), jnp.float32),
        jax.ShapeDtypeStruct((N,), jnp.float32),
    ),
    mesh=plsc.VectorSubcoreMesh(
        core_axis_name="c", subcore_axis_name="s",
        num_cores=NC, num_subcores=NS),
    compiler_params=pltpu.CompilerParams(needs_layout_passes=False),
    scratch_types=[
        pltpu.VMEM((C, D), jnp.float32),
        pltpu.VMEM((C, D), jnp.float32),
        pltpu.VMEM((C, D), jnp.float32),
        pltpu.VMEM((1, 128), jnp.int32),
        pltpu.VMEM((1, 128), jnp.int32),
        pltpu.VMEM((1, 128), jnp.int32),
        pltpu.VMEM((C,), jnp.float32),
        pltpu.VMEM((C,), jnp.float32),
        pltpu.VMEM((C,), jnp.float32),
        pltpu.VMEM((MAXM * C,), jnp.float32),
        pltpu.VMEM((32,), jnp.int32),
        pltpu.VMEM((D,), jnp.float32),
        pltpu.VMEM((16,), jnp.float32),
        pltpu.VMEM((16, D), jnp.float32),
        pltpu.VMEM_SHARED((B, D), jnp.float32),
        pltpu.SemaphoreType.DMA,
        pltpu.SemaphoreType.DMA,
        pltpu.SemaphoreType.DMA,
        pltpu.SemaphoreType.DMA,
        pltpu.SemaphoreType.DMA,
        pltpu.SemaphoreType.DMA,
        pltpu.SemaphoreType.DMA,
        pltpu.SemaphoreType.DMA,
    ],
)


def _mlp_body(p_ref, w1, b1, g1, t1, w2, b2, g2, t2, w3, b3, g3, t3,
              wp, bp, out_ref):
    gf = p_ref[0] + p_ref[1]
    dot = functools.partial(jax.lax.dot_general,
                            dimension_numbers=(((1,), (0,)), ((), ())),
                            preferred_element_type=jnp.float32,
                            precision=jax.lax.Precision.DEFAULT)
    h = jnp.maximum(dot(gf, w1[...]) + b1[...][None, :], 0.0)
    h = h * (g1[...] * _BN_INV)[None, :] + t1[...][None, :]
    h = jnp.maximum(dot(h, w2[...]) + b2[...][None, :], 0.0)
    h = h * (g2[...] * _BN_INV)[None, :] + t2[...][None, :]
    h = jnp.maximum(dot(h, w3[...]) + b3[...][None, :], 0.0)
    h = h * (g3[...] * _BN_INV)[None, :] + t3[...][None, :]
    out_ref[...] = dot(h, wp[...]) + bp[...][None, :]


_mlp_call = pl.pallas_call(
    _mlp_body,
    out_shape=jax.ShapeDtypeStruct((B, 1), jnp.float32),
)


def kernel(rgcn_node_feats, rgcn_edge_feats, smask_feats, segment_ids,
           W_aw, b_aw, W1, b1, g1, bt1, W2, b2, g2, bt2,
           W3, b3, g3, bt3, Wp, bp):
    del rgcn_edge_feats
    sm = smask_feats.reshape(N)
    waw = W_aw.reshape(D)
    baw = jnp.broadcast_to(b_aw.reshape(1), (16,))
    partial, weight = _sc_call(rgcn_node_feats, segment_ids.astype(jnp.int32),
                               sm, waw, baw)
    out = _mlp_call(partial, W1, b1, g1, bt1, W2, b2, g2, bt2,
                    W3, b3, g3, bt3, Wp, bp)
    return (out, weight.reshape(N, 1))


# speedup vs baseline: 1.0711x; 1.0711x over previous
import functools

import jax
import jax.numpy as jnp
from jax import lax
from jax.experimental import pallas as pl
from jax.experimental.pallas import tpu as pltpu
from jax.experimental.pallas import tpu_sc as plsc

N = 100000
D = 128
B = 4096
H = 256

NC = 2
NS = 16
NW = NC * NS

C = 128
FULL_CHUNKS = N // C
TAIL0 = FULL_CHUNKS * C
TAIL_ROWS = N - TAIL0
BASE_CH = FULL_CHUNKS // NW
REM_CH = FULL_CHUNKS - BASE_CH * NW
MAXM = BASE_CH + 1
NBUF = 3
_BN_INV = 1.0 / (1.0 + 1e-5) ** 0.5


def _sc_body(x_hbm, ids_hbm, sm_hbm, waw_hbm, baw_hbm,
             partial_hbm, wout_hbm,
             xv0, xv1, xv2, idv0, idv1, idv2, smv0, smv1, smv2,
             wv, idt, wawv, bawv, zv, acc,
             sin0, sin1, sin2, ssc0, ssc1, ssc2, semw, semz):
    c = lax.axis_index("c")
    s = lax.axis_index("s")
    wid = s * NC + c
    xvs = (xv0, xv1, xv2)
    idvs = (idv0, idv1, idv2)
    smvs = (smv0, smv1, smv2)
    sins = (sin0, sin1, sin2)
    sscs = (ssc0, ssc1, ssc2)

    pltpu.sync_copy(waw_hbm, wawv)
    pltpu.sync_copy(baw_hbm, bawv)

    zf = jnp.zeros((16,), jnp.float32)
    for i in range(16):
        for j in range(8):
            zv[i, pl.ds(16 * j, 16)] = zf
    for i in range(16):
        pltpu.async_copy(zv, acc.at[pl.ds(s * 256 + 16 * i, 16)], semz)

    def _copies(m, b):
        rb = (wid + NW * m) * C
        return [(x_hbm.at[pl.ds(rb, C)], xvs[b]),
                (sm_hbm.at[pl.ds(rb, C)], smvs[b]),
                (ids_hbm.at[pl.ds(rb, C)], idvs[b].at[0])]

    def _fire_in(m, b):
        for src, dst in _copies(m, b):
            pltpu.async_copy(src, dst, sins[b])

    def _wait_in(m, b):
        for src, dst in _copies(m, b):
            pltpu.make_async_copy(src, dst, sins[b]).wait()

    def _wait_sc(b):
        pltpu.make_async_copy(xvs[b], acc.at[idvs[b].at[0]], sscs[b]).wait()

    lane = lax.iota(jnp.int32, 16)

    nch = jnp.where(wid < REM_CH, 1, 0) + BASE_CH

    def _zwait(i, carry):
        pltpu.make_async_copy(zv, acc.at[pl.ds(s * 256 + 16 * i, 16)],
                              semz).wait()
        return carry
    lax.fori_loop(0, 16, _zwait, 0)
    plsc.subcore_barrier()

    _fire_in(0, 0)
    _fire_in(1, 1)

    def _process(m, b):
        _wait_in(m, b)
        xv, smv = xvs[b], smvs[b]
        ww = [wawv[pl.ds(16 * j, 16)] for j in range(8)]
        bvec = bawv[...]
        mask0 = lane == 0

        @plsc.parallel_loop(0, C, unroll=4)
        def _node_body(r):
            vj = [xv[r, pl.ds(16 * j, 16)] for j in range(8)]
            pr = [vj[j] * ww[j] for j in range(8)]
            a = ((pr[0] + pr[1]) + (pr[2] + pr[3])) \
                + ((pr[4] + pr[5]) + (pr[6] + pr[7]))
            sv = jnp.sum(a) + bvec
            sg = 1.0 / (1.0 + jnp.exp(-sv))
            smr = plsc.load_gather(smv, [jnp.full((16,), r, jnp.int32)])
            wn = sg * smr
            for j in range(8):
                xv[r, pl.ds(16 * j, 16)] = vj[j] * wn
            plsc.store_scatter(wv, [jnp.full((16,), m * C + r, jnp.int32)],
                               wn, mask=mask0)
        pltpu.async_copy(wv.at[pl.ds(m * C, C)],
                         wout_hbm.at[pl.ds((wid + NW * m) * C, C)], semw)
        nb = (m + 2) - ((m + 2) // NBUF) * NBUF

        @pl.when((m + 2 < nch) & (m >= 1))
        def _wsc():
            for bb in range(NBUF):
                @pl.when(nb == bb)
                def _w():
                    _wait_sc(bb)

        @pl.when(m + 2 < nch)
        def _pf():
            for bb in range(NBUF):
                @pl.when(nb == bb)
                def _f():
                    _fire_in(m + 2, bb)
        pltpu.async_copy(xvs[b], acc.at[idvs[b].at[0]], sscs[b], add=True)

    def _outer(k3, carry):
        for b in range(NBUF):
            m = NBUF * k3 + b

            @pl.when(m < nch)
            def _sub():
                _process(m, b)
        return carry
    lax.fori_loop(0, (MAXM + NBUF - 1) // NBUF, _outer, 0)

    @pl.when(wid < REM_CH)
    def _dr1():
        _wait_sc(1)
        _wait_sc(2)
        _wait_sc(0)

    @pl.when(wid >= REM_CH)
    def _dr2():
        _wait_sc(0)
        _wait_sc(1)
        _wait_sc(2)

    def _wdrain(m, carry):
        pltpu.make_async_copy(wv.at[pl.ds(m * C, C)],
                              wout_hbm.at[pl.ds((wid + NW * m) * C, C)],
                              semw).wait()
        return carry
    lax.fori_loop(0, nch, _wdrain, 0)

    @pl.when(wid == NW - 1)
    def _tail():
        def _zrow(r, carry):
            for j in range(8):
                xv0[r, pl.ds(16 * j, 16)] = zf
            return carry
        lax.fori_loop(TAIL_ROWS, C, _zrow, 0)
        zi = jnp.zeros((16,), jnp.int32)
        for j in range(8):
            idv0[0, pl.ds(16 * j, 16)] = zi
        pltpu.sync_copy(x_hbm.at[pl.ds(TAIL0, TAIL_ROWS)],
                        xv0.at[pl.ds(0, TAIL_ROWS)])
        pltpu.sync_copy(ids_hbm.at[pl.ds(TAIL0, TAIL_ROWS)], idt)
        for j in range(TAIL_ROWS // 16):
            idv0[0, pl.ds(16 * j, 16)] = idt[pl.ds(16 * j, 16)]
        pltpu.sync_copy(sm_hbm.at[pl.ds(TAIL0, TAIL_ROWS)],
                        smv0.at[pl.ds(0, TAIL_ROWS)])

        def _tgroup(t, carry):
            r0 = t * 16
            ww = [wawv[pl.ds(16 * j, 16)] for j in range(8)]
            bvec = bawv[...]
            smvec = smv0[pl.ds(r0, 16)]
            wvec = zf
            for i in range(16):
                r = r0 + i
                vj = [xv0[r, pl.ds(16 * j, 16)] for j in range(8)]
                pr = [vj[j] * ww[j] for j in range(8)]
                a = ((pr[0] + pr[1]) + (pr[2] + pr[3])) \
                    + ((pr[4] + pr[5]) + (pr[6] + pr[7]))
                sdot = jnp.sum(a)
                sv = sdot + bvec
                sg = 1.0 / (1.0 + jnp.exp(-sv))
                wn = sg * smvec[i]
                wvec = jnp.where(lane == i, wn, wvec)
                for j in range(8):
                    xv0[r, pl.ds(16 * j, 16)] = vj[j] * wn
            wv[pl.ds(r0, 16)] = wvec
            return carry
        lax.fori_loop(0, TAIL_ROWS // 16, _tgroup, 0)
        pltpu.sync_copy(wv.at[pl.ds(0, TAIL_ROWS)],
                        wout_hbm.at[pl.ds(TAIL0, TAIL_ROWS)])
        pltpu.sync_copy(xv0, acc.at[idv0.at[0]], add=True)

    plsc.subcore_barrier()
    pltpu.sync_copy(acc.at[pl.ds(s * 256, 128)], xv0)
    pltpu.sync_copy(acc.at[pl.ds(s * 256 + 128, 128)], xv1)
    pltpu.sync_copy(xv0, partial_hbm.at[c, pl.ds(s * 256, 128)])
    pltpu.sync_copy(xv1, partial_hbm.at[c, pl.ds(s * 256 + 128, 128)])


_sc_call = pl.kernel(
    _sc_body,
    out_type=(
        jax.ShapeDtypeStruct((NC, B, D), jnp.float32),
        jax.ShapeDtypeStruct((N,), jnp.float32),
    ),
    mesh=plsc.VectorSubcoreMesh(
        core_axis_name="c", subcore_axis_name="s",
        num_cores=NC, num_subcores=NS),
    compiler_params=pltpu.CompilerParams(needs_layout_passes=False),
    scratch_types=[
        pltpu.VMEM((C, D), jnp.float32),
        pltpu.VMEM((C, D), jnp.float32),
        pltpu.VMEM((C, D), jnp.float32),
        pltpu.VMEM((1, 128), jnp.int32),
        pltpu.VMEM((1, 128), jnp.int32),
        pltpu.VMEM((1, 128), jnp.int32),
        pltpu.VMEM((C,), jnp.float32),
        pltpu.VMEM((C,), jnp.float32),
        pltpu.VMEM((C,), jnp.float32),
        pltpu.VMEM((MAXM * C,), jnp.float32),
        pltpu.VMEM((32,), jnp.int32),
        pltpu.VMEM((D,), jnp.float32),
        pltpu.VMEM((16,), jnp.float32),
        pltpu.VMEM((16, D), jnp.float32),
        pltpu.VMEM_SHARED((B, D), jnp.float32),
        pltpu.SemaphoreType.DMA,
        pltpu.SemaphoreType.DMA,
        pltpu.SemaphoreType.DMA,
        pltpu.SemaphoreType.DMA,
        pltpu.SemaphoreType.DMA,
        pltpu.SemaphoreType.DMA,
        pltpu.SemaphoreType.DMA,
        pltpu.SemaphoreType.DMA,
    ],
)


def _mlp_body(p_ref, w1, b1, g1, t1, w2, b2, g2, t2, w3, b3, g3, t3,
              wp, bp, out_ref):
    gf = p_ref[0] + p_ref[1]
    dot = functools.partial(jax.lax.dot_general,
                            dimension_numbers=(((1,), (0,)), ((), ())),
                            preferred_element_type=jnp.float32,
                            precision=jax.lax.Precision.DEFAULT)
    h = jnp.maximum(dot(gf, w1[...]) + b1[...][None, :], 0.0)
    h = h * (g1[...] * _BN_INV)[None, :] + t1[...][None, :]
    h = jnp.maximum(dot(h, w2[...]) + b2[...][None, :], 0.0)
    h = h * (g2[...] * _BN_INV)[None, :] + t2[...][None, :]
    h = jnp.maximum(dot(h, w3[...]) + b3[...][None, :], 0.0)
    h = h * (g3[...] * _BN_INV)[None, :] + t3[...][None, :]
    out_ref[...] = dot(h, wp[...]) + bp[...][None, :]


_mlp_call = pl.pallas_call(
    _mlp_body,
    out_shape=jax.ShapeDtypeStruct((B, 1), jnp.float32),
)


def kernel(rgcn_node_feats, rgcn_edge_feats, smask_feats, segment_ids,
           W_aw, b_aw, W1, b1, g1, bt1, W2, b2, g2, bt2,
           W3, b3, g3, bt3, Wp, bp):
    del rgcn_edge_feats
    sm = smask_feats.reshape(N)
    waw = W_aw.reshape(D)
    baw = jnp.broadcast_to(b_aw.reshape(1), (16,))
    partial, weight = _sc_call(rgcn_node_feats, segment_ids.astype(jnp.int32),
                               sm, waw, baw)
    out = _mlp_call(partial, W1, b1, g1, bt1, W2, b2, g2, bt2,
                    W3, b3, g3, bt3, Wp, bp)
    return (out, weight.reshape(N, 1))


# speedup vs baseline: 1.1313x; 1.0562x over previous
import functools

import jax
import jax.numpy as jnp
from jax import lax
from jax.experimental import pallas as pl
from jax.experimental.pallas import tpu as pltpu
from jax.experimental.pallas import tpu_sc as plsc

N = 100000
D = 128
B = 4096
H = 256

NC = 2
NS = 16
NW = NC * NS

C = 128
FULL_CHUNKS = N // C
TAIL0 = FULL_CHUNKS * C
TAIL_ROWS = N - TAIL0
IDROWS = FULL_CHUNKS + 1
BASE_CH = FULL_CHUNKS // NW
REM_CH = FULL_CHUNKS - BASE_CH * NW
CPT = BASE_CH + 1
NBUF = 3
_BN_INV = 1.0 / (1.0 + 1e-5) ** 0.5


def _sc_body(x_hbm, ids_hbm, sm_hbm, waw_hbm, baw_hbm,
             partial_hbm, wout_hbm,
             xv0, xv1, xv2, idsv, smv, wv,
             xt, idtail, smt, wvt, wawv, bawv, zv, acc,
             sin0, sin1, sin2, ssc0, ssc1, ssc2,
             semw, semi, semz, semt):
    c = lax.axis_index("c")
    s = lax.axis_index("s")
    wid = s * NC + c
    xvs = (xv0, xv1, xv2)
    sins = (sin0, sin1, sin2)
    sscs = (ssc0, ssc1, ssc2)

    cw = BASE_CH * wid + jnp.minimum(wid, REM_CH)
    nch = jnp.where(wid < REM_CH, CPT, BASE_CH)

    def _xcopy(k, b):
        return (x_hbm.at[pl.ds((cw + k) * C, C)], xvs[b])

    src, dst = _xcopy(0, 0)
    pltpu.async_copy(src, dst, sin0)
    src, dst = _xcopy(1, 1)
    pltpu.async_copy(src, dst, sin1)
    for k in range(CPT):
        pltpu.async_copy(ids_hbm.at[pl.ds((cw + k) * C, C)], idsv.at[k], semi)
    pltpu.async_copy(sm_hbm.at[pl.ds(cw * C, CPT * C)], smv, semi)
    pltpu.async_copy(waw_hbm, wawv, semi)
    pltpu.async_copy(baw_hbm, bawv, semi)

    @pl.when(wid == NW - 1)
    def _tail_fire():
        pltpu.async_copy(x_hbm.at[pl.ds(TAIL0, TAIL_ROWS)],
                         xt.at[pl.ds(0, TAIL_ROWS)], semt)
        pltpu.async_copy(ids_hbm.at[pl.ds(TAIL0, C)], idtail.at[0], semt)
        pltpu.async_copy(sm_hbm.at[pl.ds(TAIL0, TAIL_ROWS)], smt, semt)

    zf = jnp.zeros((16,), jnp.float32)
    for i in range(16):
        for j in range(8):
            zv[i, pl.ds(16 * j, 16)] = zf
    for i in range(16):
        pltpu.async_copy(zv, acc.at[pl.ds(s * 256 + 16 * i, 16)], semz)

    lane = lax.iota(jnp.int32, 16)

    for k in range(CPT):
        pltpu.make_async_copy(ids_hbm.at[pl.ds((cw + k) * C, C)], idsv.at[k],
                              semi).wait()
    pltpu.make_async_copy(sm_hbm.at[pl.ds(cw * C, CPT * C)], smv, semi).wait()
    pltpu.make_async_copy(waw_hbm, wawv, semi).wait()
    pltpu.make_async_copy(baw_hbm, bawv, semi).wait()

    def _zwait(i, carry):
        pltpu.make_async_copy(zv, acc.at[pl.ds(s * 256 + 16 * i, 16)],
                              semz).wait()
        return carry
    lax.fori_loop(0, 16, _zwait, 0)
    plsc.subcore_barrier()

    def _wait_sc(b):
        pltpu.make_async_copy(xvs[b], acc.at[idsv.at[0]], sscs[b]).wait()

    def _process(k, b):
        src, dst = _xcopy(k, b)
        pltpu.make_async_copy(src, dst, sins[b]).wait()
        xv = xvs[b]

        def _group_body(t, carry):
            r0 = t * 16
            ww = [wawv[pl.ds(16 * j, 16)] for j in range(8)]
            bvec = bawv[...]
            smvec = smv[pl.ds(k * C + r0, 16)]
            wvec = zf
            for i in range(16):
                r = r0 + i
                vj = [xv[r, pl.ds(16 * j, 16)] for j in range(8)]
                pr = [vj[j] * ww[j] for j in range(8)]
                a = ((pr[0] + pr[1]) + (pr[2] + pr[3])) \
                    + ((pr[4] + pr[5]) + (pr[6] + pr[7]))
                sdot = jnp.sum(a)
                sv = sdot + bvec
                sg = 1.0 / (1.0 + jnp.exp(-sv))
                wn = sg * smvec[i]
                wvec = jnp.where(lane == i, wn, wvec)
                for j in range(8):
                    xv[r, pl.ds(16 * j, 16)] = vj[j] * wn
            wv[pl.ds(k * C + r0, 16)] = wvec
            return carry
        lax.fori_loop(0, C // 16, _group_body, 0)

        nb = (k + 2) - ((k + 2) // NBUF) * NBUF

        @pl.when((k + 2 < nch) & (k >= 1))
        def _wsc():
            for bb in range(NBUF):
                @pl.when(nb == bb)
                def _w():
                    _wait_sc(bb)

        @pl.when(k + 2 < nch)
        def _pf():
            for bb in range(NBUF):
                @pl.when(nb == bb)
                def _f():
                    src, dst = _xcopy(k + 2, bb)
                    pltpu.async_copy(src, dst, sins[bb])
        pltpu.async_copy(xvs[b], acc.at[idsv.at[k]], sscs[b], add=True)

    def _outer(k3, carry):
        for b in range(NBUF):
            k = NBUF * k3 + b

            @pl.when(k < nch)
            def _sub():
                _process(k, b)
        return carry
    lax.fori_loop(0, (CPT + NBUF - 1) // NBUF, _outer, 0)

    @pl.when(wid < REM_CH)
    def _wst1():
        pltpu.async_copy(wv, wout_hbm.at[pl.ds(cw * C, CPT * C)], semw)

    @pl.when(wid >= REM_CH)
    def _wst2():
        pltpu.async_copy(wv.at[pl.ds(0, BASE_CH * C)],
                         wout_hbm.at[pl.ds(cw * C, BASE_CH * C)], semw)

    @pl.when(wid < REM_CH)
    def _dr1():
        _wait_sc(1)
        _wait_sc(2)
        _wait_sc(0)

    @pl.when(wid >= REM_CH)
    def _dr2():
        _wait_sc(0)
        _wait_sc(1)
        _wait_sc(2)

    @pl.when(wid == NW - 1)
    def _tail():
        def _zrow(r, carry):
            for j in range(8):
                xt[r, pl.ds(16 * j, 16)] = zf
            return carry
        lax.fori_loop(TAIL_ROWS, C, _zrow, 0)
        pltpu.make_async_copy(x_hbm.at[pl.ds(TAIL0, TAIL_ROWS)],
                              xt.at[pl.ds(0, TAIL_ROWS)], semt).wait()
        pltpu.make_async_copy(ids_hbm.at[pl.ds(TAIL0, C)], idtail.at[0],
                              semt).wait()
        pltpu.make_async_copy(sm_hbm.at[pl.ds(TAIL0, TAIL_ROWS)], smt,
                              semt).wait()

        def _tgroup(t, carry):
            r0 = t * 16
            ww = [wawv[pl.ds(16 * j, 16)] for j in range(8)]
            bvec = bawv[...]
            smvec = smt[pl.ds(r0, 16)]
            wvec = zf
            for i in range(16):
                r = r0 + i
                vj = [xt[r, pl.ds(16 * j, 16)] for j in range(8)]
                pr = [vj[j] * ww[j] for j in range(8)]
                a = ((pr[0] + pr[1]) + (pr[2] + pr[3])) \
                    + ((pr[4] + pr[5]) + (pr[6] + pr[7]))
                sdot = jnp.sum(a)
                sv = sdot + bvec
                sg = 1.0 / (1.0 + jnp.exp(-sv))
                wn = sg * smvec[i]
                wvec = jnp.where(lane == i, wn, wvec)
                for j in range(8):
                    xt[r, pl.ds(16 * j, 16)] = vj[j] * wn
            wvt[pl.ds(r0, 16)] = wvec
            return carry
        lax.fori_loop(0, TAIL_ROWS // 16, _tgroup, 0)
        pltpu.sync_copy(wvt, wout_hbm.at[pl.ds(TAIL0, TAIL_ROWS)])
        pltpu.sync_copy(xt, acc.at[idtail.at[0]], add=True)

    @pl.when(wid < REM_CH)
    def _wdr1():
        pltpu.make_async_copy(wv, wout_hbm.at[pl.ds(cw * C, CPT * C)],
                              semw).wait()

    @pl.when(wid >= REM_CH)
    def _wdr2():
        pltpu.make_async_copy(wv.at[pl.ds(0, BASE_CH * C)],
                              wout_hbm.at[pl.ds(cw * C, BASE_CH * C)],
                              semw).wait()

    plsc.subcore_barrier()
    pltpu.async_copy(acc.at[pl.ds(s * 256, 128)], xv0, sin0)
    pltpu.async_copy(acc.at[pl.ds(s * 256 + 128, 128)], xv1, sin1)
    pltpu.make_async_copy(acc.at[pl.ds(s * 256, 128)], xv0, sin0).wait()
    pltpu.make_async_copy(acc.at[pl.ds(s * 256 + 128, 128)], xv1,
                          sin1).wait()
    pltpu.async_copy(xv0, partial_hbm.at[c, pl.ds(s * 256, 128)], sin0)
    pltpu.async_copy(xv1, partial_hbm.at[c, pl.ds(s * 256 + 128, 128)], sin1)
    pltpu.make_async_copy(xv0, partial_hbm.at[c, pl.ds(s * 256, 128)],
                          sin0).wait()
    pltpu.make_async_copy(xv1, partial_hbm.at[c, pl.ds(s * 256 + 128, 128)],
                          sin1).wait()


_sc_call = pl.kernel(
    _sc_body,
    out_type=(
        jax.ShapeDtypeStruct((NC, B, D), jnp.float32),
        jax.ShapeDtypeStruct((N,), jnp.float32),
    ),
    mesh=plsc.VectorSubcoreMesh(
        core_axis_name="c", subcore_axis_name="s",
        num_cores=NC, num_subcores=NS),
    compiler_params=pltpu.CompilerParams(needs_layout_passes=False),
    scratch_types=[
        pltpu.VMEM((C, D), jnp.float32),
        pltpu.VMEM((C, D), jnp.float32),
        pltpu.VMEM((C, D), jnp.float32),
        pltpu.VMEM((CPT, 128), jnp.int32),
        pltpu.VMEM((CPT * C,), jnp.float32),
        pltpu.VMEM((CPT * C,), jnp.float32),
        pltpu.VMEM((C, D), jnp.float32),
        pltpu.VMEM((1, 128), jnp.int32),
        pltpu.VMEM((TAIL_ROWS,), jnp.float32),
        pltpu.VMEM((TAIL_ROWS,), jnp.float32),
        pltpu.VMEM((D,), jnp.float32),
        pltpu.VMEM((16,), jnp.float32),
        pltpu.VMEM((16, D), jnp.float32),
        pltpu.VMEM_SHARED((B, D), jnp.float32),
        pltpu.SemaphoreType.DMA,
        pltpu.SemaphoreType.DMA,
        pltpu.SemaphoreType.DMA,
        pltpu.SemaphoreType.DMA,
        pltpu.SemaphoreType.DMA,
        pltpu.SemaphoreType.DMA,
        pltpu.SemaphoreType.DMA,
        pltpu.SemaphoreType.DMA,
        pltpu.SemaphoreType.DMA,
        pltpu.SemaphoreType.DMA,
    ],
)


def _mlp_body(p_ref, w1, b1, g1, t1, w2, b2, g2, t2, w3, b3, g3, t3,
              wp, bp, out_ref):
    gf = p_ref[0] + p_ref[1]
    dot = functools.partial(jax.lax.dot_general,
                            dimension_numbers=(((1,), (0,)), ((), ())),
                            preferred_element_type=jnp.float32,
                            precision=jax.lax.Precision.DEFAULT)
    h = jnp.maximum(dot(gf, w1[...]) + b1[...][None, :], 0.0)
    h = h * (g1[...] * _BN_INV)[None, :] + t1[...][None, :]
    h = jnp.maximum(dot(h, w2[...]) + b2[...][None, :], 0.0)
    h = h * (g2[...] * _BN_INV)[None, :] + t2[...][None, :]
    h = jnp.maximum(dot(h, w3[...]) + b3[...][None, :], 0.0)
    h = h * (g3[...] * _BN_INV)[None, :] + t3[...][None, :]
    out_ref[...] = dot(h, wp[...]) + bp[...][None, :]


_mlp_call = pl.pallas_call(
    _mlp_body,
    out_shape=jax.ShapeDtypeStruct((B, 1), jnp.float32),
)


def kernel(rgcn_node_feats, rgcn_edge_feats, smask_feats, segment_ids,
           W_aw, b_aw, W1, b1, g1, bt1, W2, b2, g2, bt2,
           W3, b3, g3, bt3, Wp, bp):
    del rgcn_edge_feats
    sm = jnp.pad(smask_feats.reshape(N), (0, IDROWS * 128 - N))
    waw = W_aw.reshape(D)
    baw = jnp.broadcast_to(b_aw.reshape(1), (16,))
    ids1 = jnp.pad(segment_ids.astype(jnp.int32), (0, IDROWS * 128 - N))
    partial, weight = _sc_call(rgcn_node_feats, ids1, sm, waw, baw)
    out = _mlp_call(partial, W1, b1, g1, bt1, W2, b2, g2, bt2,
                    W3, b3, g3, bt3, Wp, bp)
    return (out, weight.reshape(N, 1))


# speedup vs baseline: 1.1451x; 1.0122x over previous
import functools

import jax
import jax.numpy as jnp
from jax import lax
from jax.experimental import pallas as pl
from jax.experimental.pallas import tpu as pltpu
from jax.experimental.pallas import tpu_sc as plsc

N = 100000
D = 128
B = 4096
H = 256

NC = 2
NS = 16
NW = NC * NS

C = 128
FULL_CHUNKS = N // C
TAIL0 = FULL_CHUNKS * C
TAIL_ROWS = N - TAIL0
IDROWS = FULL_CHUNKS + 1
BASE_CH = FULL_CHUNKS // NW
REM_CH = FULL_CHUNKS - BASE_CH * NW
CPT = BASE_CH + 1
NBUF = 4
_BN_INV = 1.0 / (1.0 + 1e-5) ** 0.5


def _sc_body(x_hbm, ids_hbm, sm_hbm, waw_hbm, baw_hbm,
             partial_hbm, wout_hbm,
             xv0, xv1, xv2, xv3, idsv, smv, wv,
             idtail, smt, wvt, wawv, bawv, zv, acc,
             sin0, sin1, sin2, sin3, ssc0, ssc1, ssc2, ssc3,
             semw, semi, semz, semt):
    c = lax.axis_index("c")
    s = lax.axis_index("s")
    wid = s * NC + c
    xvs = (xv0, xv1, xv2, xv3)
    sins = (sin0, sin1, sin2, sin3)
    sscs = (ssc0, ssc1, ssc2, ssc3)

    cw = BASE_CH * wid + jnp.minimum(wid, REM_CH)
    nch = jnp.where(wid < REM_CH, CPT, BASE_CH)

    def _xcopy(k, b):
        return (x_hbm.at[pl.ds((cw + k) * C, C)], xvs[b])

    for k0 in range(NBUF - 1):
        src, dst = _xcopy(k0, k0)
        pltpu.async_copy(src, dst, sins[k0])
    for k in range(CPT):
        pltpu.async_copy(ids_hbm.at[pl.ds((cw + k) * C, C)], idsv.at[k], semi)
    pltpu.async_copy(sm_hbm.at[pl.ds(cw * C, CPT * C)], smv, semi)
    pltpu.async_copy(waw_hbm, wawv, semi)
    pltpu.async_copy(baw_hbm, bawv, semi)

    @pl.when(wid == NW - 1)
    def _tail_fire():
        pltpu.async_copy(ids_hbm.at[pl.ds(TAIL0, C)], idtail.at[0], semt)
        pltpu.async_copy(sm_hbm.at[pl.ds(TAIL0, TAIL_ROWS)], smt, semt)

    zf = jnp.zeros((16,), jnp.float32)
    for i in range(16):
        for j in range(8):
            zv[i, pl.ds(16 * j, 16)] = zf
    for i in range(16):
        pltpu.async_copy(zv, acc.at[pl.ds(s * 256 + 16 * i, 16)], semz)

    lane = lax.iota(jnp.int32, 16)

    for k in range(CPT):
        pltpu.make_async_copy(ids_hbm.at[pl.ds((cw + k) * C, C)], idsv.at[k],
                              semi).wait()
    pltpu.make_async_copy(sm_hbm.at[pl.ds(cw * C, CPT * C)], smv, semi).wait()
    pltpu.make_async_copy(waw_hbm, wawv, semi).wait()
    pltpu.make_async_copy(baw_hbm, bawv, semi).wait()

    def _zwait(i, carry):
        pltpu.make_async_copy(zv, acc.at[pl.ds(s * 256 + 16 * i, 16)],
                              semz).wait()
        return carry
    lax.fori_loop(0, 16, _zwait, 0)
    plsc.subcore_barrier()

    def _wait_sc(b):
        pltpu.make_async_copy(xvs[b], acc.at[idsv.at[0]], sscs[b]).wait()

    def _process(k, b):
        src, dst = _xcopy(k, b)
        pltpu.make_async_copy(src, dst, sins[b]).wait()
        xv = xvs[b]

        def _group_body(t, carry):
            r0 = t * 16
            ww = [wawv[pl.ds(16 * j, 16)] for j in range(8)]
            bvec = bawv[...]
            smvec = smv[pl.ds(k * C + r0, 16)]
            wvec = zf
            for i in range(16):
                r = r0 + i
                vj = [xv[r, pl.ds(16 * j, 16)] for j in range(8)]
                pr = [vj[j] * ww[j] for j in range(8)]
                a = ((pr[0] + pr[1]) + (pr[2] + pr[3])) \
                    + ((pr[4] + pr[5]) + (pr[6] + pr[7]))
                sdot = jnp.sum(a)
                sv = sdot + bvec
                sg = 1.0 / (1.0 + jnp.exp(-sv))
                wn = sg * smvec[i]
                wvec = jnp.where(lane == i, wn, wvec)
                for j in range(8):
                    xv[r, pl.ds(16 * j, 16)] = vj[j] * wn
            wv[pl.ds(k * C + r0, 16)] = wvec
            return carry
        lax.fori_loop(0, C // 16, _group_body, 0)

        nb = (k + 3) - ((k + 3) // NBUF) * NBUF

        @pl.when((k + 3 < nch) & (k >= 1))
        def _wsc():
            for bb in range(NBUF):
                @pl.when(nb == bb)
                def _w():
                    _wait_sc(bb)

        @pl.when(k + 3 < nch)
        def _pf():
            for bb in range(NBUF):
                @pl.when(nb == bb)
                def _f():
                    src, dst = _xcopy(k + 3, bb)
                    pltpu.async_copy(src, dst, sins[bb])
        pltpu.async_copy(xvs[b], acc.at[idsv.at[k]], sscs[b], add=True)

    def _outer(k3, carry):
        for b in range(NBUF):
            k = NBUF * k3 + b

            @pl.when(k < nch)
            def _sub():
                _process(k, b)
        return carry
    lax.fori_loop(0, (CPT + NBUF - 1) // NBUF, _outer, 0)

    @pl.when(wid < REM_CH)
    def _wst1():
        pltpu.async_copy(wv, wout_hbm.at[pl.ds(cw * C, CPT * C)], semw)

    @pl.when(wid >= REM_CH)
    def _wst2():
        pltpu.async_copy(wv.at[pl.ds(0, BASE_CH * C)],
                         wout_hbm.at[pl.ds(cw * C, BASE_CH * C)], semw)

    @pl.when(wid < REM_CH)
    def _dr1():
        _wait_sc(1)
        _wait_sc(2)
        _wait_sc(3)
        _wait_sc(0)

    @pl.when(wid >= REM_CH)
    def _dr2():
        _wait_sc(0)
        _wait_sc(1)
        _wait_sc(2)
        _wait_sc(3)

    @pl.when(wid == NW - 1)
    def _tail():
        pltpu.async_copy(x_hbm.at[pl.ds(TAIL0, TAIL_ROWS)],
                         xv3.at[pl.ds(0, TAIL_ROWS)], semt)

        def _zrow(r, carry):
            for j in range(8):
                xv3[r, pl.ds(16 * j, 16)] = zf
            return carry
        lax.fori_loop(TAIL_ROWS, C, _zrow, 0)
        pltpu.make_async_copy(x_hbm.at[pl.ds(TAIL0, TAIL_ROWS)],
                              xv3.at[pl.ds(0, TAIL_ROWS)], semt).wait()
        pltpu.make_async_copy(ids_hbm.at[pl.ds(TAIL0, C)], idtail.at[0],
                              semt).wait()
        pltpu.make_async_copy(sm_hbm.at[pl.ds(TAIL0, TAIL_ROWS)], smt,
                              semt).wait()

        def _tgroup(t, carry):
            r0 = t * 16
            ww = [wawv[pl.ds(16 * j, 16)] for j in range(8)]
            bvec = bawv[...]
            smvec = smt[pl.ds(r0, 16)]
            wvec = zf
            for i in range(16):
                r = r0 + i
                vj = [xv3[r, pl.ds(16 * j, 16)] for j in range(8)]
                pr = [vj[j] * ww[j] for j in range(8)]
                a = ((pr[0] + pr[1]) + (pr[2] + pr[3])) \
                    + ((pr[4] + pr[5]) + (pr[6] + pr[7]))
                sdot = jnp.sum(a)
                sv = sdot + bvec
                sg = 1.0 / (1.0 + jnp.exp(-sv))
                wn = sg * smvec[i]
                wvec = jnp.where(lane == i, wn, wvec)
                for j in range(8):
                    xv3[r, pl.ds(16 * j, 16)] = vj[j] * wn
            wvt[pl.ds(r0, 16)] = wvec
            return carry
        lax.fori_loop(0, TAIL_ROWS // 16, _tgroup, 0)
        pltpu.sync_copy(wvt, wout_hbm.at[pl.ds(TAIL0, TAIL_ROWS)])
        pltpu.sync_copy(xv3, acc.at[idtail.at[0]], add=True)

    @pl.when(wid < REM_CH)
    def _wdr1():
        pltpu.make_async_copy(wv, wout_hbm.at[pl.ds(cw * C, CPT * C)],
                              semw).wait()

    @pl.when(wid >= REM_CH)
    def _wdr2():
        pltpu.make_async_copy(wv.at[pl.ds(0, BASE_CH * C)],
                              wout_hbm.at[pl.ds(cw * C, BASE_CH * C)],
                              semw).wait()

    plsc.subcore_barrier()
    pltpu.async_copy(acc.at[pl.ds(s * 256, 128)], xv0, sin0)
    pltpu.async_copy(acc.at[pl.ds(s * 256 + 128, 128)], xv1, sin1)
    pltpu.make_async_copy(acc.at[pl.ds(s * 256, 128)], xv0, sin0).wait()
    pltpu.make_async_copy(acc.at[pl.ds(s * 256 + 128, 128)], xv1,
                          sin1).wait()
    pltpu.async_copy(xv0, partial_hbm.at[c, pl.ds(s * 256, 128)], sin0)
    pltpu.async_copy(xv1, partial_hbm.at[c, pl.ds(s * 256 + 128, 128)], sin1)
    pltpu.make_async_copy(xv0, partial_hbm.at[c, pl.ds(s * 256, 128)],
                          sin0).wait()
    pltpu.make_async_copy(xv1, partial_hbm.at[c, pl.ds(s * 256 + 128, 128)],
                          sin1).wait()


_sc_call = pl.kernel(
    _sc_body,
    out_type=(
        jax.ShapeDtypeStruct((NC, B, D), jnp.float32),
        jax.ShapeDtypeStruct((N,), jnp.float32),
    ),
    mesh=plsc.VectorSubcoreMesh(
        core_axis_name="c", subcore_axis_name="s",
        num_cores=NC, num_subcores=NS),
    compiler_params=pltpu.CompilerParams(needs_layout_passes=False),
    scratch_types=[
        pltpu.VMEM((C, D), jnp.float32),
        pltpu.VMEM((C, D), jnp.float32),
        pltpu.VMEM((C, D), jnp.float32),
        pltpu.VMEM((C, D), jnp.float32),
        pltpu.VMEM((CPT, 128), jnp.int32),
        pltpu.VMEM((CPT * C,), jnp.float32),
        pltpu.VMEM((CPT * C,), jnp.float32),
        pltpu.VMEM((1, 128), jnp.int32),
        pltpu.VMEM((TAIL_ROWS,), jnp.float32),
        pltpu.VMEM((TAIL_ROWS,), jnp.float32),
        pltpu.VMEM((D,), jnp.float32),
        pltpu.VMEM((16,), jnp.float32),
        pltpu.VMEM((16, D), jnp.float32),
        pltpu.VMEM_SHARED((B, D), jnp.float32),
        pltpu.SemaphoreType.DMA,
        pltpu.SemaphoreType.DMA,
        pltpu.SemaphoreType.DMA,
        pltpu.SemaphoreType.DMA,
        pltpu.SemaphoreType.DMA,
        pltpu.SemaphoreType.DMA,
        pltpu.SemaphoreType.DMA,
        pltpu.SemaphoreType.DMA,
        pltpu.SemaphoreType.DMA,
        pltpu.SemaphoreType.DMA,
        pltpu.SemaphoreType.DMA,
        pltpu.SemaphoreType.DMA,
    ],
)


def _mlp_body(p_ref, w1, b1, g1, t1, w2, b2, g2, t2, w3, b3, g3, t3,
              wp, bp, out_ref):
    gf = p_ref[0] + p_ref[1]
    dot = functools.partial(jax.lax.dot_general,
                            dimension_numbers=(((1,), (0,)), ((), ())),
                            preferred_element_type=jnp.float32,
                            precision=jax.lax.Precision.DEFAULT)
    h = jnp.maximum(dot(gf, w1[...]) + b1[...][None, :], 0.0)
    h = h * (g1[...] * _BN_INV)[None, :] + t1[...][None, :]
    h = jnp.maximum(dot(h, w2[...]) + b2[...][None, :], 0.0)
    h = h * (g2[...] * _BN_INV)[None, :] + t2[...][None, :]
    h = jnp.maximum(dot(h, w3[...]) + b3[...][None, :], 0.0)
    h = h * (g3[...] * _BN_INV)[None, :] + t3[...][None, :]
    out_ref[...] = dot(h, wp[...]) + bp[...][None, :]


_mlp_call = pl.pallas_call(
    _mlp_body,
    out_shape=jax.ShapeDtypeStruct((B, 1), jnp.float32),
)


def kernel(rgcn_node_feats, rgcn_edge_feats, smask_feats, segment_ids,
           W_aw, b_aw, W1, b1, g1, bt1, W2, b2, g2, bt2,
           W3, b3, g3, bt3, Wp, bp):
    del rgcn_edge_feats
    sm = jnp.pad(smask_feats.reshape(N), (0, IDROWS * 128 - N))
    waw = W_aw.reshape(D)
    baw = jnp.broadcast_to(b_aw.reshape(1), (16,))
    ids1 = jnp.pad(segment_ids.astype(jnp.int32), (0, IDROWS * 128 - N))
    partial, weight = _sc_call(rgcn_node_feats, ids1, sm, waw, baw)
    out = _mlp_call(partial, W1, b1, g1, bt1, W2, b2, g2, bt2,
                    W3, b3, g3, bt3, Wp, bp)
    return (out, weight.reshape(N, 1))
